# baseline (speedup 1.0000x reference)
import jax
import jax.numpy as jnp
from jax.experimental import pallas as pl
from jax.experimental.pallas import tpu as pltpu

_N_USERS = 100_000
_N_ITEMS = 1_000_000
_RANK = 32
_BLOCK = 65536
_GRID = (_N_ITEMS + _BLOCK - 1) // _BLOCK


def _score_body(uid_ref, ut_ref, vt_ref, out_ref, u_scratch, u_sem):
    uid = uid_ref[0]
    col0 = pl.multiple_of((uid // 128) * 128, 128)
    c = uid % 128

    @pl.when(pl.program_id(0) == 0)
    def _fetch_user_tile():
        copy = pltpu.make_async_copy(
            ut_ref.at[:, pl.ds(col0, 128)], u_scratch, u_sem
        )
        copy.start()
        copy.wait()

    lane = jax.lax.broadcasted_iota(jnp.int32, (_RANK, 128), 1)
    u_col = jnp.sum(
        jnp.where(lane == c, u_scratch[...], 0.0), axis=1, keepdims=True
    )
    scores = jax.lax.dot_general(
        u_col,
        vt_ref[...],
        dimension_numbers=(((0,), (0,)), ((), ())),
        preferred_element_type=jnp.float32,
    )
    out_ref[...] = scores.reshape((_BLOCK,))


def kernel(user_id, U, V):
    uid = jnp.asarray(user_id, jnp.int32).reshape((1,))
    ut = U.T
    vt = V.T
    grid_spec = pltpu.PrefetchScalarGridSpec(
        num_scalar_prefetch=1,
        grid=(_GRID,),
        in_specs=[
            pl.BlockSpec(memory_space=pltpu.HBM),
            pl.BlockSpec((_RANK, _BLOCK), lambda i, uid_ref: (0, i)),
        ],
        out_specs=pl.BlockSpec((_BLOCK,), lambda i, uid_ref: (i,)),
        scratch_shapes=[
            pltpu.VMEM((_RANK, 128), jnp.float32),
            pltpu.SemaphoreType.DMA,
        ],
    )
    return pl.pallas_call(
        _score_body,
        grid_spec=grid_spec,
        out_shape=jax.ShapeDtypeStruct((_N_ITEMS,), jnp.float32),
    )(uid, ut, vt)
